# Initial kernel scaffold; baseline (speedup 1.0000x reference)
import jax
import jax.numpy as jnp
from jax.experimental import pallas as pl


def kernel(input_ids, token_embed_weight):
    raise NotImplementedError("write your pallas kernel here")


# speedup vs baseline: 1.1028x; 1.1028x over previous
import functools
import math

import jax
import jax.numpy as jnp
import numpy as np
from jax import lax
from jax.experimental import pallas as pl
from jax.experimental.pallas import tpu as pltpu
from jax.experimental.pallas import tpu_sc as plsc

VOCAB = 100000
HIDDEN = 1024
ROT = 128
BASE_LEN = 2048
STAGE1 = 4096
MAXLEN = 8192
THETA = 10000.0
SCALE = math.sqrt(HIDDEN)

NC = 2
NS = 16
L = 16
NW = NC * NS
FHALF = ROT // 2

CHUNK = 64


def _find_correction_dim(num_rotations, dim, base, max_pos):
    return (dim * math.log(max_pos / (num_rotations * 2.0 * math.pi))) / (
        2.0 * math.log(base))


def _yarn_scale_np(inv_freq, scale, orig_len, beta_fast=32.0, beta_slow=1.0):
    dim_half = inv_freq.shape[0]
    low = max(math.floor(_find_correction_dim(beta_fast, ROT, THETA, orig_len)), 0)
    high = min(math.ceil(_find_correction_dim(beta_slow, ROT, THETA, orig_len)),
               dim_half - 1)
    ramp = np.clip(
        (np.arange(dim_half, dtype=np.float32) - low) / max(high - low, 1e-3),
        0.0, 1.0).astype(np.float32)
    extrap_mask = (1.0 - ramp).astype(np.float32)
    inv_freq_interp = (inv_freq / np.float32(scale)).astype(np.float32)
    return (inv_freq_interp * (1.0 - extrap_mask)
            + inv_freq * extrap_mask).astype(np.float32)


def _inv_freq_np(target_len):
    inv_freq = (1.0 / (np.float32(THETA) ** (
        np.arange(0, ROT, 2, dtype=np.float32) / np.float32(ROT)))).astype(
            np.float32)
    if target_len > BASE_LEN:
        inv_freq = _yarn_scale_np(inv_freq, float(STAGE1) / float(BASE_LEN),
                                  BASE_LEN)
    if target_len > STAGE1:
        inv_freq = _yarn_scale_np(inv_freq, float(MAXLEN) / float(STAGE1),
                                  STAGE1)
    return inv_freq


def _make_sc_call(n_tok, seq_len):
    assert n_tok % NW == 0 and seq_len % NW == 0
    tok_per_w = n_tok // NW
    assert tok_per_w % CHUNK == 0
    n_chunks = tok_per_w // CHUNK
    frows = seq_len // NW

    mesh = plsc.VectorSubcoreMesh(core_axis_name="c", subcore_axis_name="s")

    @functools.partial(
        pl.kernel,
        mesh=mesh,
        out_type=[
            jax.ShapeDtypeStruct((n_tok, HIDDEN), jnp.float32),
            jax.ShapeDtypeStruct((seq_len, FHALF), jnp.float32),
        ],
        scratch_types=[
            pltpu.VMEM((CHUNK,), jnp.int32),
            pltpu.VMEM((CHUNK, HIDDEN), jnp.float32),
            pltpu.VMEM((frows, FHALF), jnp.float32),
            pltpu.VMEM((FHALF,), jnp.float32),
            pltpu.SemaphoreType.DMA,
        ],
    )
    def sc_call(ids_hbm, table_hbm, invf_hbm, x_hbm, fr_hbm,
                idx_v, rows_v, fr_v, invf_v, sem):
        wid = lax.axis_index("s") * NC + lax.axis_index("c")

        pltpu.sync_copy(invf_hbm, invf_v)
        fbase = wid * frows

        def frow(r, carry):
            t = (fbase + r).astype(jnp.float32)
            for j in range(FHALF // L):
                sl = pl.ds(j * L, L)
                fr_v[r, sl] = invf_v[sl] * t
            return carry

        lax.fori_loop(0, frows, frow, None)
        pltpu.sync_copy(fr_v, fr_hbm.at[pl.ds(fbase, frows)])

        tbase = wid * tok_per_w

        def chunk(g, carry):
            base = tbase + g * CHUNK
            pltpu.sync_copy(ids_hbm.at[pl.ds(base, CHUNK)], idx_v)
            pltpu.async_copy(table_hbm.at[idx_v], rows_v, sem).wait()

            def row(r, c2):
                def vec(j, c3):
                    sl = pl.ds(j * L, L)
                    rows_v[r, sl] = rows_v[r, sl] * SCALE
                    return c3
                lax.fori_loop(0, HIDDEN // L, vec, None, unroll=8)
                return c2

            lax.fori_loop(0, CHUNK, row, None)
            pltpu.sync_copy(rows_v, x_hbm.at[pl.ds(base, CHUNK)])
            return carry

        lax.fori_loop(0, n_chunks, chunk, None)

    return sc_call


def kernel(input_ids, token_embed_weight):
    batch, seq_len = input_ids.shape
    n_tok = batch * seq_len
    ids_flat = input_ids.reshape(n_tok)
    invf = jnp.asarray(_inv_freq_np(seq_len))
    sc_call = _make_sc_call(n_tok, seq_len)
    x2, freqs = sc_call(ids_flat, token_embed_weight, invf)
    return x2.reshape(batch, seq_len, HIDDEN), freqs


# speedup vs baseline: 1.3571x; 1.2306x over previous
import functools
import math

import jax
import jax.numpy as jnp
import numpy as np
from jax import lax
from jax.experimental import pallas as pl
from jax.experimental.pallas import tpu as pltpu
from jax.experimental.pallas import tpu_sc as plsc

VOCAB = 100000
HIDDEN = 1024
ROT = 128
BASE_LEN = 2048
STAGE1 = 4096
MAXLEN = 8192
THETA = 10000.0
SCALE = math.sqrt(HIDDEN)

NC = 2
NS = 16
L = 16
NW = NC * NS
FHALF = ROT // 2

CHUNK = 32


def _find_correction_dim(num_rotations, dim, base, max_pos):
    return (dim * math.log(max_pos / (num_rotations * 2.0 * math.pi))) / (
        2.0 * math.log(base))


def _yarn_scale_np(inv_freq, scale, orig_len, beta_fast=32.0, beta_slow=1.0):
    dim_half = inv_freq.shape[0]
    low = max(math.floor(_find_correction_dim(beta_fast, ROT, THETA, orig_len)), 0)
    high = min(math.ceil(_find_correction_dim(beta_slow, ROT, THETA, orig_len)),
               dim_half - 1)
    ramp = np.clip(
        (np.arange(dim_half, dtype=np.float32) - low) / max(high - low, 1e-3),
        0.0, 1.0).astype(np.float32)
    extrap_mask = (1.0 - ramp).astype(np.float32)
    inv_freq_interp = (inv_freq / np.float32(scale)).astype(np.float32)
    return (inv_freq_interp * (1.0 - extrap_mask)
            + inv_freq * extrap_mask).astype(np.float32)


def _inv_freq_np(target_len):
    inv_freq = (1.0 / (np.float32(THETA) ** (
        np.arange(0, ROT, 2, dtype=np.float32) / np.float32(ROT)))).astype(
            np.float32)
    if target_len > BASE_LEN:
        inv_freq = _yarn_scale_np(inv_freq, float(STAGE1) / float(BASE_LEN),
                                  BASE_LEN)
    if target_len > STAGE1:
        inv_freq = _yarn_scale_np(inv_freq, float(MAXLEN) / float(STAGE1),
                                  STAGE1)
    return inv_freq


def _make_sc_call(n_tok, seq_len):
    assert n_tok % NW == 0 and seq_len % NW == 0
    tok_per_w = n_tok // NW
    assert tok_per_w % CHUNK == 0
    n_chunks = tok_per_w // CHUNK
    frows = seq_len // NW

    mesh = plsc.VectorSubcoreMesh(core_axis_name="c", subcore_axis_name="s")

    @functools.partial(
        pl.kernel,
        mesh=mesh,
        out_type=[
            jax.ShapeDtypeStruct((n_tok, HIDDEN), jnp.float32),
            jax.ShapeDtypeStruct((seq_len, FHALF), jnp.float32),
        ],
        scratch_types=[
            pltpu.VMEM((CHUNK,), jnp.int32),
            pltpu.VMEM((CHUNK,), jnp.int32),
            pltpu.VMEM((CHUNK, HIDDEN), jnp.float32),
            pltpu.VMEM((CHUNK, HIDDEN), jnp.float32),
            pltpu.VMEM((frows, FHALF), jnp.float32),
            pltpu.VMEM((FHALF,), jnp.float32),
            pltpu.SemaphoreType.DMA,
            pltpu.SemaphoreType.DMA,
            pltpu.SemaphoreType.DMA,
            pltpu.SemaphoreType.DMA,
        ],
    )
    def sc_call(ids_hbm, table_hbm, invf_hbm, x_hbm, fr_hbm,
                idx0, idx1, rows0, rows1, fr_v, invf_v,
                gsem0, gsem1, ssem0, ssem1):
        wid = lax.axis_index("s") * NC + lax.axis_index("c")
        idx = (idx0, idx1)
        rows = (rows0, rows1)
        gsem = (gsem0, gsem1)
        ssem = (ssem0, ssem1)
        tbase = wid * tok_per_w

        def scale_rows(rv):
            def row(r, c2):
                def vec(j, c3):
                    sl = pl.ds(j * L, L)
                    rv[r, sl] = rv[r, sl] * SCALE
                    return c3
                lax.fori_loop(0, HIDDEN // L, vec, None, unroll=8)
                return c2
            lax.fori_loop(0, CHUNK, row, None)

        pltpu.sync_copy(ids_hbm.at[pl.ds(tbase, CHUNK)], idx[0])
        gd = [pltpu.async_copy(table_hbm.at[idx[0]], rows[0], gsem[0]), None]

        pltpu.sync_copy(invf_hbm, invf_v)
        fbase = wid * frows

        def frow(r, carry):
            t = (fbase + r).astype(jnp.float32)
            for j in range(FHALF // L):
                sl = pl.ds(j * L, L)
                fr_v[r, sl] = invf_v[sl] * t
            return carry

        lax.fori_loop(0, frows, frow, None)
        pltpu.sync_copy(fr_v, fr_hbm.at[pl.ds(fbase, frows)])

        sd = [None, None]
        for g in range(n_chunks):
            b = g & 1
            if g + 1 < n_chunks:
                ob = (g + 1) & 1
                if sd[ob] is not None:
                    sd[ob].wait()
                pltpu.sync_copy(
                    ids_hbm.at[pl.ds(tbase + (g + 1) * CHUNK, CHUNK)], idx[ob])
                gd[ob] = pltpu.async_copy(
                    table_hbm.at[idx[ob]], rows[ob], gsem[ob])
            gd[b].wait()
            scale_rows(rows[b])
            sd[b] = pltpu.async_copy(
                rows[b], x_hbm.at[pl.ds(tbase + g * CHUNK, CHUNK)], ssem[b])
        sd[0].wait()
        sd[1].wait()

    return sc_call


def kernel(input_ids, token_embed_weight):
    batch, seq_len = input_ids.shape
    n_tok = batch * seq_len
    ids_flat = input_ids.reshape(n_tok)
    invf = jnp.asarray(_inv_freq_np(seq_len))
    sc_call = _make_sc_call(n_tok, seq_len)
    x2, freqs = sc_call(ids_flat, token_embed_weight, invf)
    return x2.reshape(batch, seq_len, HIDDEN), freqs


# speedup vs baseline: 1.4673x; 1.0812x over previous
import functools
import math

import jax
import jax.numpy as jnp
import numpy as np
from jax import lax
from jax.experimental import pallas as pl
from jax.experimental.pallas import tpu as pltpu
from jax.experimental.pallas import tpu_sc as plsc

VOCAB = 100000
HIDDEN = 1024
ROT = 128
BASE_LEN = 2048
STAGE1 = 4096
MAXLEN = 8192
THETA = 10000.0
SCALE = math.sqrt(HIDDEN)

NC = 2
NS = 16
L = 16
NW = NC * NS
FHALF = ROT // 2

CHUNK = 32
NBUF = 3


def _find_correction_dim(num_rotations, dim, base, max_pos):
    return (dim * math.log(max_pos / (num_rotations * 2.0 * math.pi))) / (
        2.0 * math.log(base))


def _yarn_scale_np(inv_freq, scale, orig_len, beta_fast=32.0, beta_slow=1.0):
    dim_half = inv_freq.shape[0]
    low = max(math.floor(_find_correction_dim(beta_fast, ROT, THETA, orig_len)), 0)
    high = min(math.ceil(_find_correction_dim(beta_slow, ROT, THETA, orig_len)),
               dim_half - 1)
    ramp = np.clip(
        (np.arange(dim_half, dtype=np.float32) - low) / max(high - low, 1e-3),
        0.0, 1.0).astype(np.float32)
    extrap_mask = (1.0 - ramp).astype(np.float32)
    inv_freq_interp = (inv_freq / np.float32(scale)).astype(np.float32)
    return (inv_freq_interp * (1.0 - extrap_mask)
            + inv_freq * extrap_mask).astype(np.float32)


def _inv_freq_np(target_len):
    inv_freq = (1.0 / (np.float32(THETA) ** (
        np.arange(0, ROT, 2, dtype=np.float32) / np.float32(ROT)))).astype(
            np.float32)
    if target_len > BASE_LEN:
        inv_freq = _yarn_scale_np(inv_freq, float(STAGE1) / float(BASE_LEN),
                                  BASE_LEN)
    if target_len > STAGE1:
        inv_freq = _yarn_scale_np(inv_freq, float(MAXLEN) / float(STAGE1),
                                  STAGE1)
    return inv_freq


def _make_sc_call(batch, seq_len):
    n_tok = batch * seq_len
    assert n_tok % NW == 0 and seq_len % NW == 0
    tok_per_w = n_tok // NW
    spans_per_row = seq_len // tok_per_w
    assert tok_per_w % CHUNK == 0
    n_chunks = tok_per_w // CHUNK
    frows = seq_len // NW

    mesh = plsc.VectorSubcoreMesh(core_axis_name="c", subcore_axis_name="s")

    @functools.partial(
        pl.kernel,
        mesh=mesh,
        out_type=[
            jax.ShapeDtypeStruct((batch, seq_len, HIDDEN), jnp.float32),
            jax.ShapeDtypeStruct((seq_len, FHALF), jnp.float32),
        ],
        scratch_types=[
            pltpu.VMEM((tok_per_w,), jnp.int32),
            pltpu.VMEM((CHUNK, HIDDEN), jnp.float32),
            pltpu.VMEM((CHUNK, HIDDEN), jnp.float32),
            pltpu.VMEM((CHUNK, HIDDEN), jnp.float32),
            pltpu.VMEM((frows, FHALF), jnp.float32),
            pltpu.VMEM((FHALF,), jnp.float32),
            pltpu.SemaphoreType.DMA,
            pltpu.SemaphoreType.DMA,
            pltpu.SemaphoreType.DMA,
            pltpu.SemaphoreType.DMA,
            pltpu.SemaphoreType.DMA,
            pltpu.SemaphoreType.DMA,
        ],
    )
    def sc_call(ids_hbm, table_hbm, invf_hbm, x_hbm, fr_hbm,
                idx_all, rows0, rows1, rows2, fr_v, invf_v,
                gsem0, gsem1, gsem2, ssem0, ssem1, ssem2):
        wid = lax.axis_index("s") * NC + lax.axis_index("c")
        rows = (rows0, rows1, rows2)
        gsem = (gsem0, gsem1, gsem2)
        ssem = (ssem0, ssem1, ssem2)
        bidx = wid // spans_per_row
        soff = (wid % spans_per_row) * tok_per_w

        pltpu.sync_copy(ids_hbm.at[bidx, pl.ds(soff, tok_per_w)], idx_all)

        def gather(g):
            return pltpu.async_copy(
                table_hbm.at[idx_all.at[pl.ds(g * CHUNK, CHUNK)]],
                rows[g % NBUF], gsem[g % NBUF])

        def scatter(g):
            return pltpu.async_copy(
                rows[g % NBUF],
                x_hbm.at[bidx, pl.ds(soff + g * CHUNK, CHUNK)],
                ssem[g % NBUF])

        def scale_rows(rv):
            def row(r, c2):
                def vec(j, c3):
                    sl = pl.ds(j * L, L)
                    rv[r, sl] = rv[r, sl] * SCALE
                    return c3
                lax.fori_loop(0, HIDDEN // L, vec, None, unroll=8)
                return c2
            lax.fori_loop(0, CHUNK, row, None)

        gd = [None] * NBUF
        sd = [None] * NBUF
        gd[0] = gather(0)
        gd[1] = gather(1)

        pltpu.sync_copy(invf_hbm, invf_v)
        fbase = wid * frows

        def frow(r, carry):
            t = (fbase + r).astype(jnp.float32)
            for j in range(FHALF // L):
                sl = pl.ds(j * L, L)
                fr_v[r, sl] = invf_v[sl] * t
            return carry

        lax.fori_loop(0, frows, frow, None)
        pltpu.sync_copy(fr_v, fr_hbm.at[pl.ds(fbase, frows)])

        for g in range(n_chunks):
            b = g % NBUF
            gd[b].wait()
            scale_rows(rows[b])
            sd[b] = scatter(g)
            nxt = g + 2
            if nxt < n_chunks:
                nb = nxt % NBUF
                if sd[nb] is not None:
                    sd[nb].wait()
                gd[nb] = gather(nxt)
        for g in range(max(0, n_chunks - NBUF), n_chunks):
            sd[g % NBUF].wait()

    return sc_call


def kernel(input_ids, token_embed_weight):
    batch, seq_len = input_ids.shape
    invf = jnp.asarray(_inv_freq_np(seq_len))
    sc_call = _make_sc_call(batch, seq_len)
    x, freqs = sc_call(input_ids, token_embed_weight, invf)
    return x, freqs


# speedup vs baseline: 1.5045x; 1.0254x over previous
import functools
import math

import jax
import jax.numpy as jnp
import numpy as np
from jax import lax
from jax.experimental import pallas as pl
from jax.experimental.pallas import tpu as pltpu
from jax.experimental.pallas import tpu_sc as plsc

VOCAB = 100000
HIDDEN = 1024
ROT = 128
BASE_LEN = 2048
STAGE1 = 4096
MAXLEN = 8192
THETA = 10000.0
SCALE = math.sqrt(HIDDEN)

NC = 2
NS = 16
L = 16
NW = NC * NS
FHALF = ROT // 2

CHUNK = 32
NBUF = 3


def _find_correction_dim(num_rotations, dim, base, max_pos):
    return (dim * math.log(max_pos / (num_rotations * 2.0 * math.pi))) / (
        2.0 * math.log(base))


def _yarn_scale_np(inv_freq, scale, orig_len, beta_fast=32.0, beta_slow=1.0):
    dim_half = inv_freq.shape[0]
    low = max(math.floor(_find_correction_dim(beta_fast, ROT, THETA, orig_len)), 0)
    high = min(math.ceil(_find_correction_dim(beta_slow, ROT, THETA, orig_len)),
               dim_half - 1)
    ramp = np.clip(
        (np.arange(dim_half, dtype=np.float32) - low) / max(high - low, 1e-3),
        0.0, 1.0).astype(np.float32)
    extrap_mask = (1.0 - ramp).astype(np.float32)
    inv_freq_interp = (inv_freq / np.float32(scale)).astype(np.float32)
    return (inv_freq_interp * (1.0 - extrap_mask)
            + inv_freq * extrap_mask).astype(np.float32)


def _inv_freq_np(target_len):
    inv_freq = (1.0 / (np.float32(THETA) ** (
        np.arange(0, ROT, 2, dtype=np.float32) / np.float32(ROT)))).astype(
            np.float32)
    if target_len > BASE_LEN:
        inv_freq = _yarn_scale_np(inv_freq, float(STAGE1) / float(BASE_LEN),
                                  BASE_LEN)
    if target_len > STAGE1:
        inv_freq = _yarn_scale_np(inv_freq, float(MAXLEN) / float(STAGE1),
                                  STAGE1)
    return inv_freq


def _make_sc_call(batch, seq_len):
    n_tok = batch * seq_len
    assert n_tok % NW == 0 and seq_len % NW == 0
    tok_per_w = n_tok // NW
    spans_per_row = seq_len // tok_per_w
    assert tok_per_w % CHUNK == 0
    n_chunks = tok_per_w // CHUNK
    frows = seq_len // NW

    mesh = plsc.VectorSubcoreMesh(core_axis_name="c", subcore_axis_name="s")

    @functools.partial(
        pl.kernel,
        mesh=mesh,
        out_type=[
            jax.ShapeDtypeStruct((batch, seq_len, HIDDEN), jnp.float32),
            jax.ShapeDtypeStruct((seq_len, FHALF), jnp.float32),
        ],
        scratch_types=[
            pltpu.VMEM((tok_per_w,), jnp.int32),
            pltpu.VMEM((CHUNK, HIDDEN), jnp.float32),
            pltpu.VMEM((CHUNK, HIDDEN), jnp.float32),
            pltpu.VMEM((CHUNK, HIDDEN), jnp.float32),
            pltpu.VMEM((frows, FHALF), jnp.float32),
            pltpu.VMEM((FHALF,), jnp.float32),
            pltpu.SemaphoreType.DMA,
            pltpu.SemaphoreType.DMA,
            pltpu.SemaphoreType.DMA,
            pltpu.SemaphoreType.DMA,
            pltpu.SemaphoreType.DMA,
            pltpu.SemaphoreType.DMA,
        ],
    )
    def sc_call(ids_hbm, table_hbm, invf_hbm, x_hbm, fr_hbm,
                idx_all, rows0, rows1, rows2, fr_v, invf_v,
                gsem0, gsem1, gsem2, ssem0, ssem1, ssem2):
        wid = lax.axis_index("s") * NC + lax.axis_index("c")
        rows = (rows0, rows1, rows2)
        gsem = (gsem0, gsem1, gsem2)
        ssem = (ssem0, ssem1, ssem2)
        bidx = wid // spans_per_row
        soff = (wid % spans_per_row) * tok_per_w

        pltpu.sync_copy(ids_hbm.at[bidx, pl.ds(soff, tok_per_w)], idx_all)

        def gather(g):
            return pltpu.async_copy(
                table_hbm.at[idx_all.at[pl.ds(g * CHUNK, CHUNK)]],
                rows[g % NBUF], gsem[g % NBUF])

        def scatter(g):
            return pltpu.async_copy(
                rows[g % NBUF],
                x_hbm.at[bidx, pl.ds(soff + g * CHUNK, CHUNK)],
                ssem[g % NBUF])

        def scale_rows(rv):
            def row(r, c2):
                def vec(j, c3):
                    sl = pl.ds(j * L, L)
                    rv[r, sl] = rv[r, sl] * SCALE
                    return c3
                lax.fori_loop(0, HIDDEN // L, vec, None, unroll=8)
                return c2
            lax.fori_loop(0, CHUNK, row, None)

        gd = [None] * NBUF
        sd = [None] * NBUF
        gd[0] = gather(0)
        gd[1] = gather(1)

        pltpu.sync_copy(invf_hbm, invf_v)
        fbase = wid * frows

        def frow(r, carry):
            t = (fbase + r).astype(jnp.float32)
            for j in range(FHALF // L):
                sl = pl.ds(j * L, L)
                fr_v[r, sl] = invf_v[sl] * t
            return carry

        lax.fori_loop(0, frows, frow, None)
        pltpu.sync_copy(fr_v, fr_hbm.at[pl.ds(fbase, frows)])

        gd[0].wait()
        scale_rows(rows[0])
        sd[0] = scatter(0)
        gd[2] = gather(2)
        gd[1].wait()
        scale_rows(rows[1])
        sd[1] = scatter(1)
        sd[0].wait()
        gd[0] = gather(3)

        def block(g0):
            for b in range(NBUF):
                g = g0 + b
                bb = (2 + b) % NBUF
                nb = (2 + b + 2) % NBUF
                pltpu.make_async_copy(
                    table_hbm.at[idx_all.at[pl.ds(g * CHUNK, CHUNK)]],
                    rows[bb], gsem[bb]).wait()
                scale_rows(rows[bb])
                pltpu.async_copy(
                    rows[bb],
                    x_hbm.at[bidx, pl.ds(soff + g * CHUNK, CHUNK)],
                    ssem[bb])
                pltpu.make_async_copy(
                    rows[nb],
                    x_hbm.at[bidx, pl.ds(soff + (g - 1) * CHUNK, CHUNK)],
                    ssem[nb]).wait()
                pltpu.async_copy(
                    table_hbm.at[idx_all.at[pl.ds((g + 2) * CHUNK, CHUNK)]],
                    rows[nb], gsem[nb])

        assert (n_chunks - 2 - 2) % NBUF == 0
        pl.loop(2, n_chunks - 4, step=NBUF)(block)

        for g in (n_chunks - 2, n_chunks - 1):
            b = g % NBUF
            pltpu.make_async_copy(
                table_hbm.at[idx_all.at[pl.ds(g * CHUNK, CHUNK)]],
                rows[b], gsem[b]).wait()
            scale_rows(rows[b])
            pltpu.async_copy(
                rows[b],
                x_hbm.at[bidx, pl.ds(soff + g * CHUNK, CHUNK)],
                ssem[b])
        for g in range(n_chunks - NBUF, n_chunks):
            b = g % NBUF
            pltpu.make_async_copy(
                rows[b],
                x_hbm.at[bidx, pl.ds(soff + g * CHUNK, CHUNK)],
                ssem[b]).wait()

    return sc_call


def kernel(input_ids, token_embed_weight):
    batch, seq_len = input_ids.shape
    invf = jnp.asarray(_inv_freq_np(seq_len))
    sc_call = _make_sc_call(batch, seq_len)
    x, freqs = sc_call(input_ids, token_embed_weight, invf)
    return x, freqs
